# baseline (speedup 1.0000x reference)
import functools

import jax
import jax.numpy as jnp
from jax import lax
from jax.experimental import pallas as pl
from jax.experimental.pallas import tpu as pltpu
from jax.experimental.pallas import tpu_sc as plsc

_DIM = 1024
_E = 8
_INNER = 4096
_T = 2048
_B = 128
_NBLK = _T // _B + _E
_P = _NBLK * _B
_EPS = 1e-06

_NC = 2
_NS = 16
_NW = _NC * _NS
_BPW = _T // _NW


def _gate_body(x_ref, wg_ref, bg_ref, xs_ref, p_ref, bmap_ref, act_ref):
    x = x_ref[...]
    logits = jnp.dot(x, wg_ref[...], preferred_element_type=jnp.float32)
    logits = logits + bg_ref[...]
    gs = jax.nn.softmax(logits, axis=-1)
    mx = jnp.max(gs, axis=1, keepdims=True)
    lane = lax.broadcasted_iota(jnp.int32, (_T, _E), 1)
    eid = jnp.min(jnp.where(gs == mx, lane, _E), axis=1, keepdims=True)
    onehot = (lane == eid).astype(jnp.float32)

    denom = jnp.sum(gs * onehot, axis=0, keepdims=True)
    counts_i = jnp.sum(onehot, axis=0, keepdims=True).astype(jnp.int32)
    pc = ((counts_i + (_B - 1)) // _B) * _B
    pc_f = pc.astype(jnp.float32)

    r8 = lax.broadcasted_iota(jnp.int32, (_E, _E), 0)
    c8 = lax.broadcasted_iota(jnp.int32, (_E, _E), 1)
    upper = (r8 < c8).astype(jnp.float32)
    off = jnp.dot(pc_f, upper, preferred_element_type=jnp.float32)

    r_t = lax.broadcasted_iota(jnp.int32, (_T, _T), 0)
    c_t = lax.broadcasted_iota(jnp.int32, (_T, _T), 1)
    ltri = (r_t > c_t).astype(jnp.float32)
    cum = jnp.dot(ltri, onehot, preferred_element_type=jnp.float32)
    rank = jnp.sum(cum * onehot, axis=1, keepdims=True)

    off_t = jnp.sum(off * onehot, axis=1, keepdims=True)
    denom_t = jnp.sum(denom * onehot, axis=1, keepdims=True)
    coeff = mx / (denom_t + _EPS) * float(_T)
    p_ref[...] = (off_t + rank).astype(jnp.int32)
    xs_ref[...] = x * coeff

    ends = off + pc_f
    rows_f = (lax.broadcasted_iota(jnp.int32, (_NBLK, _E), 0) * _B
              ).astype(jnp.float32)
    bmap_raw = jnp.sum((rows_f >= ends).astype(jnp.int32), axis=1,
                       keepdims=True)
    total = jnp.sum(pc)
    blk_lo = lax.broadcasted_iota(jnp.int32, (_NBLK, 1), 0) * _B
    act = (blk_lo < total).astype(jnp.int32)
    max_e = jnp.max(jnp.where(act == 1, bmap_raw, 0))
    bmap_ref[...] = jnp.minimum(bmap_raw, max_e)
    act_ref[...] = act


_KT = 2
_IK = _INNER // _KT
_S = _KT * _NBLK
_NSPLIT = 4


def _ffn_body(sched_ref, act_ref, x_ref, w1_hbm, w2_hbm, out_ref,
              acc_ref, w1buf, w2buf, w1sem, w2sem):
    k = pl.program_id(0)
    i = pl.program_id(1)
    s = k * _NBLK + i
    chg = sched_ref[0, s]
    slot = sched_ref[1, s]
    cur_e = sched_ref[2, s]
    cur_k = sched_ref[3, s]
    nxt_e = sched_ref[4, s]
    nxt_k = sched_ref[5, s]
    has_n = sched_ref[6, s]
    nxt2_e = sched_ref[7, s]
    nxt2_k = sched_ref[8, s]
    has_n2 = sched_ref[9, s]
    slot2 = sched_ref[10, s]

    def copies(e, kk, sl):
        cs = []
        r1 = _DIM // _NSPLIT
        r2 = _IK // _NSPLIT
        for j in range(_NSPLIT):
            cs.append(pltpu.make_async_copy(
                w1_hbm.at[e, pl.ds(j * r1, r1), pl.ds(kk * _IK, _IK)],
                w1buf.at[sl, pl.ds(j * r1, r1)], w1sem.at[sl, j]))
            cs.append(pltpu.make_async_copy(
                w2_hbm.at[e, pl.ds(kk * _IK + j * r2, r2), :],
                w2buf.at[sl, pl.ds(j * r2, r2)], w2sem.at[sl, j]))
        return cs

    @pl.when(s == 0)
    def _():
        for c in copies(cur_e, cur_k, slot):
            c.start()

        @pl.when(has_n == 1)
        def _():
            for c in copies(nxt_e, nxt_k, 1):
                c.start()

    @pl.when(chg == 1)
    def _():
        for c in copies(cur_e, cur_k, slot):
            c.wait()

        @pl.when(has_n2 == 1)
        def _():
            for c in copies(nxt2_e, nxt2_k, slot2):
                c.start()

    @pl.when(act_ref[i] == 1)
    def _():
        h = jnp.dot(x_ref[...], w1buf[slot],
                    preferred_element_type=jnp.float32)
        h = jnp.maximum(h, 0.0)
        part = jnp.dot(h, w2buf[slot], preferred_element_type=jnp.float32)
        sl = pl.ds(i * _B, _B)

        @pl.when(k == 0)
        def _():
            acc_ref[sl, :] = part.astype(jnp.bfloat16)

        @pl.when(k == _KT - 1)
        def _():
            out_ref[...] = acc_ref[sl, :].astype(jnp.float32) + part


@functools.cache
def _make_sc_kernels():
    mesh = plsc.VectorSubcoreMesh(core_axis_name="c", subcore_axis_name="s")
    scratch = [
        pltpu.VMEM((_BPW,), jnp.int32),
        pltpu.VMEM((_BPW, _DIM), jnp.float32),
        pltpu.SemaphoreType.DMA,
    ]

    @functools.partial(
        pl.kernel,
        mesh=mesh,
        out_type=jax.ShapeDtypeStruct((_P, _DIM), jnp.float32),
        scratch_types=scratch,
    )
    def sc_scatter(x_hbm, p_hbm, out_hbm, idx_v, rows_v, sem):
        wid = lax.axis_index("s") * _NC + lax.axis_index("c")
        base = wid * _BPW
        pltpu.sync_copy(p_hbm.at[pl.ds(base, _BPW)], idx_v)
        pltpu.sync_copy(x_hbm.at[pl.ds(base, _BPW)], rows_v)
        pltpu.async_copy(rows_v, out_hbm.at[idx_v], sem).wait()

    @functools.partial(
        pl.kernel,
        mesh=mesh,
        out_type=jax.ShapeDtypeStruct((_T, _DIM), jnp.float32),
        scratch_types=scratch,
    )
    def sc_gather(src_hbm, p_hbm, out_hbm, idx_v, rows_v, sem):
        wid = lax.axis_index("s") * _NC + lax.axis_index("c")
        base = wid * _BPW
        pltpu.sync_copy(p_hbm.at[pl.ds(base, _BPW)], idx_v)
        pltpu.async_copy(src_hbm.at[idx_v], rows_v, sem).wait()
        pltpu.sync_copy(rows_v, out_hbm.at[pl.ds(base, _BPW)])

    return sc_scatter, sc_gather


def kernel(x, Wg, bg, W1, W2):
    batch, n, dim = x.shape
    x_flat = x.reshape((_T, _DIM))

    xs, p, bmap, act = pl.pallas_call(
        _gate_body,
        out_shape=(
            jax.ShapeDtypeStruct((_T, _DIM), jnp.float32),
            jax.ShapeDtypeStruct((_T, 1), jnp.int32),
            jax.ShapeDtypeStruct((_NBLK, 1), jnp.int32),
            jax.ShapeDtypeStruct((_NBLK, 1), jnp.int32),
        ),
    )(x_flat, Wg, bg.reshape((1, _E)))

    sc_scatter, sc_gather = _make_sc_kernels()
    p_flat = p.reshape((_T,))
    x_sorted = sc_scatter(xs, p_flat)

    bmap_flat = bmap.reshape((_NBLK,))
    e_step = jnp.tile(bmap_flat, _KT)
    k_step = jnp.repeat(jnp.arange(_KT, dtype=jnp.int32), _NBLK)
    idx = jnp.arange(_S, dtype=jnp.int32)
    prev_e = jnp.roll(e_step, 1)
    prev_k = jnp.roll(k_step, 1)
    chg = jnp.where(
        (idx == 0) | (e_step != prev_e) | (k_step != prev_k), 1, 0
    ).astype(jnp.int32)
    seg = jnp.cumsum(chg) - 1
    slot = (seg % 3).astype(jnp.int32)
    slot2 = ((seg + 2) % 3).astype(jnp.int32)
    chg_idx = jnp.where(chg == 1, idx, _S)
    nc = jnp.min(
        jnp.where(idx[None, :] > idx[:, None], chg_idx[None, :], _S), axis=1)
    has_n = (nc < _S).astype(jnp.int32)
    nc_c = jnp.minimum(nc, _S - 1)
    nc2 = jnp.where(has_n == 1, nc[nc_c], _S)
    has_n2 = (nc2 < _S).astype(jnp.int32)
    nc2_c = jnp.minimum(nc2, _S - 1)
    sched = jnp.stack([
        chg, slot, e_step, k_step, e_step[nc_c], k_step[nc_c], has_n,
        e_step[nc2_c], k_step[nc2_c], has_n2, slot2,
    ]).astype(jnp.int32)

    grid_spec = pltpu.PrefetchScalarGridSpec(
        num_scalar_prefetch=2,
        grid=(_KT, _NBLK),
        in_specs=[
            pl.BlockSpec((_B, _DIM), lambda k, i, sc, ac: (i, 0)),
            pl.BlockSpec(memory_space=pltpu.MemorySpace.HBM),
            pl.BlockSpec(memory_space=pltpu.MemorySpace.HBM),
        ],
        out_specs=pl.BlockSpec(
            (_B, _DIM), lambda k, i, sc, ac: (jnp.where(k == _KT - 1, i, 0), 0)),
        scratch_shapes=[
            pltpu.VMEM((_P, _DIM), jnp.bfloat16),
            pltpu.VMEM((3, _DIM, _IK), jnp.float32),
            pltpu.VMEM((3, _IK, _DIM), jnp.float32),
            pltpu.SemaphoreType.DMA((3, _NSPLIT)),
            pltpu.SemaphoreType.DMA((3, _NSPLIT)),
        ],
    )
    ffn_sorted = pl.pallas_call(
        _ffn_body,
        grid_spec=grid_spec,
        out_shape=jax.ShapeDtypeStruct((_P, _DIM), jnp.float32),
    )(sched, act.reshape((_NBLK,)), x_sorted, W1, W2)

    out = sc_gather(ffn_sorted, p_flat)
    return out.reshape((batch, n, dim))


# speedup vs baseline: 1.0451x; 1.0451x over previous
import functools

import jax
import jax.numpy as jnp
from jax import lax
from jax.experimental import pallas as pl
from jax.experimental.pallas import tpu as pltpu
from jax.experimental.pallas import tpu_sc as plsc

_DIM = 1024
_E = 8
_INNER = 4096
_T = 2048
_B = 128
_NBLK = _T // _B + _E
_P = _NBLK * _B
_EPS = 1e-06

_NC = 2
_NS = 16
_NW = _NC * _NS
_BPW = _T // _NW


def _gate_body(x_ref, wg_ref, bg_ref, xs_ref, p_ref, bmap_ref, act_ref):
    x = x_ref[...]
    logits = jnp.dot(x, wg_ref[...], preferred_element_type=jnp.float32)
    logits = logits + bg_ref[...]
    gs = jax.nn.softmax(logits, axis=-1)
    mx = jnp.max(gs, axis=1, keepdims=True)
    lane = lax.broadcasted_iota(jnp.int32, (_T, _E), 1)
    eid = jnp.min(jnp.where(gs == mx, lane, _E), axis=1, keepdims=True)
    onehot = (lane == eid).astype(jnp.float32)

    denom = jnp.sum(gs * onehot, axis=0, keepdims=True)
    counts_i = jnp.sum(onehot, axis=0, keepdims=True).astype(jnp.int32)
    pc = ((counts_i + (_B - 1)) // _B) * _B
    pc_f = pc.astype(jnp.float32)

    r8 = lax.broadcasted_iota(jnp.int32, (_E, _E), 0)
    c8 = lax.broadcasted_iota(jnp.int32, (_E, _E), 1)
    upper = (r8 < c8).astype(jnp.float32)
    off = jnp.dot(pc_f, upper, preferred_element_type=jnp.float32)

    r_t = lax.broadcasted_iota(jnp.int32, (_T, _T), 0)
    c_t = lax.broadcasted_iota(jnp.int32, (_T, _T), 1)
    ltri = (r_t > c_t).astype(jnp.float32)
    cum = jnp.dot(ltri, onehot, preferred_element_type=jnp.float32)
    rank = jnp.sum(cum * onehot, axis=1, keepdims=True)

    off_t = jnp.sum(off * onehot, axis=1, keepdims=True)
    denom_t = jnp.sum(denom * onehot, axis=1, keepdims=True)
    coeff = mx / (denom_t + _EPS) * float(_T)
    p_ref[...] = (off_t + rank).astype(jnp.int32)
    xs_ref[...] = x * coeff

    ends = off + pc_f
    rows_f = (lax.broadcasted_iota(jnp.int32, (_NBLK, _E), 0) * _B
              ).astype(jnp.float32)
    bmap_raw = jnp.sum((rows_f >= ends).astype(jnp.int32), axis=1,
                       keepdims=True)
    total = jnp.sum(pc)
    blk_lo = lax.broadcasted_iota(jnp.int32, (_NBLK, 1), 0) * _B
    act = (blk_lo < total).astype(jnp.int32)
    max_e = jnp.max(jnp.where(act == 1, bmap_raw, 0))
    bmap_ref[...] = jnp.minimum(bmap_raw, max_e)
    act_ref[...] = act


_KT = 2
_IK = _INNER // _KT
_S = _KT * _NBLK
_NSPLIT = 4


def _ffn_body(sched_ref, act_ref, x_ref, w1_hbm, w2_hbm, out_ref,
              acc_ref, w1buf, w2buf, w1sem, w2sem):
    k = pl.program_id(0)
    i = pl.program_id(1)
    s = k * _NBLK + i
    chg = sched_ref[0, s]
    slot = sched_ref[1, s]
    cur_e = sched_ref[2, s]
    cur_k = sched_ref[3, s]
    nxt_e = sched_ref[4, s]
    nxt_k = sched_ref[5, s]
    has_n = sched_ref[6, s]

    def copies(e, kk, sl):
        cs = []
        r1 = _DIM // _NSPLIT
        r2 = _IK // _NSPLIT
        for j in range(_NSPLIT):
            cs.append(pltpu.make_async_copy(
                w1_hbm.at[e, pl.ds(j * r1, r1), pl.ds(kk * _IK, _IK)],
                w1buf.at[sl, pl.ds(j * r1, r1)], w1sem.at[sl, j]))
            cs.append(pltpu.make_async_copy(
                w2_hbm.at[e, pl.ds(kk * _IK + j * r2, r2), :],
                w2buf.at[sl, pl.ds(j * r2, r2)], w2sem.at[sl, j]))
        return cs

    @pl.when(s == 0)
    def _():
        for c in copies(cur_e, cur_k, slot):
            c.start()

    @pl.when(chg == 1)
    def _():
        for c in copies(cur_e, cur_k, slot):
            c.wait()

        @pl.when(has_n == 1)
        def _():
            for c in copies(nxt_e, nxt_k, 1 - slot):
                c.start()

    @pl.when(act_ref[i] == 1)
    def _():
        h = jnp.dot(x_ref[...], w1buf[slot],
                    preferred_element_type=jnp.float32)
        h = jnp.maximum(h, 0.0)
        part = jnp.dot(h, w2buf[slot], preferred_element_type=jnp.float32)
        sl = pl.ds(i * _B, _B)

        @pl.when(k == 0)
        def _():
            acc_ref[sl, :] = part

        @pl.when(k != 0)
        def _():
            acc_ref[sl, :] = acc_ref[sl, :] + part

        @pl.when(k == _KT - 1)
        def _():
            out_ref[...] = acc_ref[sl, :]


@functools.cache
def _make_sc_kernels():
    mesh = plsc.VectorSubcoreMesh(core_axis_name="c", subcore_axis_name="s")
    scratch = [
        pltpu.VMEM((_BPW,), jnp.int32),
        pltpu.VMEM((_BPW, _DIM), jnp.float32),
        pltpu.SemaphoreType.DMA,
    ]

    @functools.partial(
        pl.kernel,
        mesh=mesh,
        out_type=jax.ShapeDtypeStruct((_P, _DIM), jnp.float32),
        scratch_types=scratch,
    )
    def sc_scatter(x_hbm, p_hbm, out_hbm, idx_v, rows_v, sem):
        wid = lax.axis_index("s") * _NC + lax.axis_index("c")
        base = wid * _BPW
        pltpu.sync_copy(p_hbm.at[pl.ds(base, _BPW)], idx_v)
        pltpu.sync_copy(x_hbm.at[pl.ds(base, _BPW)], rows_v)
        pltpu.async_copy(rows_v, out_hbm.at[idx_v], sem).wait()

    @functools.partial(
        pl.kernel,
        mesh=mesh,
        out_type=jax.ShapeDtypeStruct((_T, _DIM), jnp.float32),
        scratch_types=scratch,
    )
    def sc_gather(src_hbm, p_hbm, out_hbm, idx_v, rows_v, sem):
        wid = lax.axis_index("s") * _NC + lax.axis_index("c")
        base = wid * _BPW
        pltpu.sync_copy(p_hbm.at[pl.ds(base, _BPW)], idx_v)
        pltpu.async_copy(src_hbm.at[idx_v], rows_v, sem).wait()
        pltpu.sync_copy(rows_v, out_hbm.at[pl.ds(base, _BPW)])

    return sc_scatter, sc_gather


def kernel(x, Wg, bg, W1, W2):
    batch, n, dim = x.shape
    x_flat = x.reshape((_T, _DIM))

    xs, p, bmap, act = pl.pallas_call(
        _gate_body,
        out_shape=(
            jax.ShapeDtypeStruct((_T, _DIM), jnp.float32),
            jax.ShapeDtypeStruct((_T, 1), jnp.int32),
            jax.ShapeDtypeStruct((_NBLK, 1), jnp.int32),
            jax.ShapeDtypeStruct((_NBLK, 1), jnp.int32),
        ),
    )(x_flat, Wg, bg.reshape((1, _E)))

    sc_scatter, sc_gather = _make_sc_kernels()
    p_flat = p.reshape((_T,))
    x_sorted = sc_scatter(xs, p_flat)

    bmap_flat = bmap.reshape((_NBLK,))
    e_step = jnp.tile(bmap_flat, _KT)
    k_step = jnp.repeat(jnp.arange(_KT, dtype=jnp.int32), _NBLK)
    idx = jnp.arange(_S, dtype=jnp.int32)
    prev_e = jnp.roll(e_step, 1)
    prev_k = jnp.roll(k_step, 1)
    chg = jnp.where(
        (idx == 0) | (e_step != prev_e) | (k_step != prev_k), 1, 0
    ).astype(jnp.int32)
    seg = jnp.cumsum(chg) - 1
    slot = (seg % 2).astype(jnp.int32)
    chg_idx = jnp.where(chg == 1, idx, _S)
    nc = jnp.min(
        jnp.where(idx[None, :] > idx[:, None], chg_idx[None, :], _S), axis=1)
    has_n = (nc < _S).astype(jnp.int32)
    nc_c = jnp.minimum(nc, _S - 1)
    sched = jnp.stack([
        chg, slot, e_step, k_step, e_step[nc_c], k_step[nc_c], has_n,
    ]).astype(jnp.int32)

    grid_spec = pltpu.PrefetchScalarGridSpec(
        num_scalar_prefetch=2,
        grid=(_KT, _NBLK),
        in_specs=[
            pl.BlockSpec(
                (_B, _DIM),
                lambda k, i, sc, ac: (jnp.where(ac[i] == 1, i, 0), 0)),
            pl.BlockSpec(memory_space=pltpu.MemorySpace.HBM),
            pl.BlockSpec(memory_space=pltpu.MemorySpace.HBM),
        ],
        out_specs=pl.BlockSpec(
            (_B, _DIM), lambda k, i, sc, ac: (jnp.where(k == _KT - 1, i, 0), 0)),
        scratch_shapes=[
            pltpu.VMEM((_P, _DIM), jnp.float32),
            pltpu.VMEM((2, _DIM, _IK), jnp.float32),
            pltpu.VMEM((2, _IK, _DIM), jnp.float32),
            pltpu.SemaphoreType.DMA((2, _NSPLIT)),
            pltpu.SemaphoreType.DMA((2, _NSPLIT)),
        ],
    )
    ffn_sorted = pl.pallas_call(
        _ffn_body,
        grid_spec=grid_spec,
        out_shape=jax.ShapeDtypeStruct((_P, _DIM), jnp.float32),
    )(sched, act.reshape((_NBLK,)), x_sorted, W1, W2)

    out = sc_gather(ffn_sorted, p_flat)
    return out.reshape((batch, n, dim))


# speedup vs baseline: 1.0682x; 1.0221x over previous
import functools

import jax
import jax.numpy as jnp
from jax import lax
from jax.experimental import pallas as pl
from jax.experimental.pallas import tpu as pltpu
from jax.experimental.pallas import tpu_sc as plsc

_DIM = 1024
_E = 8
_INNER = 4096
_T = 2048
_B = 128
_NBLK = _T // _B + _E
_P = _NBLK * _B
_EPS = 1e-06

_NC = 2
_NS = 16
_NW = _NC * _NS
_BPW = _T // _NW


def _gate_body(x_ref, wg_ref, bg_ref, xs_ref, p_ref, bmap_ref, act_ref):
    x = x_ref[...]
    logits = jnp.dot(x, wg_ref[...], preferred_element_type=jnp.float32)
    logits = logits + bg_ref[...]
    gs = jax.nn.softmax(logits, axis=-1)
    mx = jnp.max(gs, axis=1, keepdims=True)
    lane = lax.broadcasted_iota(jnp.int32, (_T, _E), 1)
    eid = jnp.min(jnp.where(gs == mx, lane, _E), axis=1, keepdims=True)
    onehot = (lane == eid).astype(jnp.float32)

    denom = jnp.sum(gs * onehot, axis=0, keepdims=True)
    counts_i = jnp.sum(onehot, axis=0, keepdims=True).astype(jnp.int32)
    pc = ((counts_i + (_B - 1)) // _B) * _B
    pc_f = pc.astype(jnp.float32)

    r8 = lax.broadcasted_iota(jnp.int32, (_E, _E), 0)
    c8 = lax.broadcasted_iota(jnp.int32, (_E, _E), 1)
    upper = (r8 < c8).astype(jnp.float32)
    off = jnp.dot(pc_f, upper, preferred_element_type=jnp.float32)

    ex = jnp.concatenate(
        [jnp.zeros((1, _E), jnp.float32), onehot[:-1]], axis=0)
    sh = 1
    while sh < _T:
        ex = ex + jnp.concatenate(
            [jnp.zeros((sh, _E), jnp.float32), ex[:-sh]], axis=0)
        sh *= 2
    rank = jnp.sum(ex * onehot, axis=1, keepdims=True)

    off_t = jnp.sum(off * onehot, axis=1, keepdims=True)
    denom_t = jnp.sum(denom * onehot, axis=1, keepdims=True)
    coeff = mx / (denom_t + _EPS) * float(_T)
    p_ref[...] = (off_t + rank).astype(jnp.int32)
    xs_ref[...] = x * coeff

    ends = off + pc_f
    rows_f = (lax.broadcasted_iota(jnp.int32, (_NBLK, _E), 0) * _B
              ).astype(jnp.float32)
    bmap_raw = jnp.sum((rows_f >= ends).astype(jnp.int32), axis=1,
                       keepdims=True)
    total = jnp.sum(pc)
    blk_lo = lax.broadcasted_iota(jnp.int32, (_NBLK, 1), 0) * _B
    act = (blk_lo < total).astype(jnp.int32)
    max_e = jnp.max(jnp.where(act == 1, bmap_raw, 0))
    bmap_ref[...] = jnp.minimum(bmap_raw, max_e)
    act_ref[...] = act


_KT = 2
_IK = _INNER // _KT
_S = _KT * _NBLK
_NSPLIT = 4


def _ffn_body(sched_ref, act_ref, x_ref, w1_hbm, w2_hbm, out_ref,
              acc_ref, w1buf, w2buf, w1sem, w2sem):
    k = pl.program_id(0)
    i = pl.program_id(1)
    s = k * _NBLK + i
    chg = sched_ref[0, s]
    slot = sched_ref[1, s]
    cur_e = sched_ref[2, s]
    cur_k = sched_ref[3, s]
    nxt_e = sched_ref[4, s]
    nxt_k = sched_ref[5, s]
    has_n = sched_ref[6, s]

    def copies(e, kk, sl):
        cs = []
        r1 = _DIM // _NSPLIT
        r2 = _IK // _NSPLIT
        for j in range(_NSPLIT):
            cs.append(pltpu.make_async_copy(
                w1_hbm.at[e, pl.ds(j * r1, r1), pl.ds(kk * _IK, _IK)],
                w1buf.at[sl, pl.ds(j * r1, r1)], w1sem.at[sl, j]))
            cs.append(pltpu.make_async_copy(
                w2_hbm.at[e, pl.ds(kk * _IK + j * r2, r2), :],
                w2buf.at[sl, pl.ds(j * r2, r2)], w2sem.at[sl, j]))
        return cs

    @pl.when(s == 0)
    def _():
        for c in copies(cur_e, cur_k, slot):
            c.start()

    @pl.when(chg == 1)
    def _():
        for c in copies(cur_e, cur_k, slot):
            c.wait()

        @pl.when(has_n == 1)
        def _():
            for c in copies(nxt_e, nxt_k, 1 - slot):
                c.start()

    @pl.when(act_ref[i] == 1)
    def _():
        h = jnp.dot(x_ref[...], w1buf[slot],
                    preferred_element_type=jnp.float32)
        h = jnp.maximum(h, 0.0)
        part = jnp.dot(h, w2buf[slot], preferred_element_type=jnp.float32)
        sl = pl.ds(i * _B, _B)

        @pl.when(k == 0)
        def _():
            acc_ref[sl, :] = part

        @pl.when(k != 0)
        def _():
            acc_ref[sl, :] = acc_ref[sl, :] + part

        @pl.when(k == _KT - 1)
        def _():
            out_ref[...] = acc_ref[sl, :]


@functools.cache
def _make_sc_kernels():
    mesh = plsc.VectorSubcoreMesh(core_axis_name="c", subcore_axis_name="s")
    scratch = [
        pltpu.VMEM((_BPW,), jnp.int32),
        pltpu.VMEM((_BPW, _DIM), jnp.float32),
        pltpu.SemaphoreType.DMA,
    ]

    @functools.partial(
        pl.kernel,
        mesh=mesh,
        out_type=jax.ShapeDtypeStruct((_P, _DIM), jnp.float32),
        scratch_types=scratch,
    )
    def sc_scatter(x_hbm, p_hbm, out_hbm, idx_v, rows_v, sem):
        wid = lax.axis_index("s") * _NC + lax.axis_index("c")
        base = wid * _BPW
        pltpu.sync_copy(p_hbm.at[pl.ds(base, _BPW)], idx_v)
        pltpu.sync_copy(x_hbm.at[pl.ds(base, _BPW)], rows_v)
        pltpu.async_copy(rows_v, out_hbm.at[idx_v], sem).wait()

    @functools.partial(
        pl.kernel,
        mesh=mesh,
        out_type=jax.ShapeDtypeStruct((_T, _DIM), jnp.float32),
        scratch_types=scratch,
    )
    def sc_gather(src_hbm, p_hbm, out_hbm, idx_v, rows_v, sem):
        wid = lax.axis_index("s") * _NC + lax.axis_index("c")
        base = wid * _BPW
        pltpu.sync_copy(p_hbm.at[pl.ds(base, _BPW)], idx_v)
        pltpu.async_copy(src_hbm.at[idx_v], rows_v, sem).wait()
        pltpu.sync_copy(rows_v, out_hbm.at[pl.ds(base, _BPW)])

    return sc_scatter, sc_gather


def kernel(x, Wg, bg, W1, W2):
    batch, n, dim = x.shape
    x_flat = x.reshape((_T, _DIM))

    xs, p, bmap, act = pl.pallas_call(
        _gate_body,
        out_shape=(
            jax.ShapeDtypeStruct((_T, _DIM), jnp.float32),
            jax.ShapeDtypeStruct((_T, 1), jnp.int32),
            jax.ShapeDtypeStruct((_NBLK, 1), jnp.int32),
            jax.ShapeDtypeStruct((_NBLK, 1), jnp.int32),
        ),
    )(x_flat, Wg, bg.reshape((1, _E)))

    sc_scatter, sc_gather = _make_sc_kernels()
    p_flat = p.reshape((_T,))
    x_sorted = sc_scatter(xs, p_flat)

    bmap_flat = bmap.reshape((_NBLK,))
    e_step = jnp.tile(bmap_flat, _KT)
    k_step = jnp.repeat(jnp.arange(_KT, dtype=jnp.int32), _NBLK)
    idx = jnp.arange(_S, dtype=jnp.int32)
    prev_e = jnp.roll(e_step, 1)
    prev_k = jnp.roll(k_step, 1)
    chg = jnp.where(
        (idx == 0) | (e_step != prev_e) | (k_step != prev_k), 1, 0
    ).astype(jnp.int32)
    seg = jnp.cumsum(chg) - 1
    slot = (seg % 2).astype(jnp.int32)
    chg_idx = jnp.where(chg == 1, idx, _S)
    nc = jnp.min(
        jnp.where(idx[None, :] > idx[:, None], chg_idx[None, :], _S), axis=1)
    has_n = (nc < _S).astype(jnp.int32)
    nc_c = jnp.minimum(nc, _S - 1)
    sched = jnp.stack([
        chg, slot, e_step, k_step, e_step[nc_c], k_step[nc_c], has_n,
    ]).astype(jnp.int32)

    grid_spec = pltpu.PrefetchScalarGridSpec(
        num_scalar_prefetch=2,
        grid=(_KT, _NBLK),
        in_specs=[
            pl.BlockSpec(
                (_B, _DIM),
                lambda k, i, sc, ac: (jnp.where(ac[i] == 1, i, 0), 0)),
            pl.BlockSpec(memory_space=pltpu.MemorySpace.HBM),
            pl.BlockSpec(memory_space=pltpu.MemorySpace.HBM),
        ],
        out_specs=pl.BlockSpec(
            (_B, _DIM), lambda k, i, sc, ac: (jnp.where(k == _KT - 1, i, 0), 0)),
        scratch_shapes=[
            pltpu.VMEM((_P, _DIM), jnp.float32),
            pltpu.VMEM((2, _DIM, _IK), jnp.float32),
            pltpu.VMEM((2, _IK, _DIM), jnp.float32),
            pltpu.SemaphoreType.DMA((2, _NSPLIT)),
            pltpu.SemaphoreType.DMA((2, _NSPLIT)),
        ],
    )
    ffn_sorted = pl.pallas_call(
        _ffn_body,
        grid_spec=grid_spec,
        out_shape=jax.ShapeDtypeStruct((_P, _DIM), jnp.float32),
    )(sched, act.reshape((_NBLK,)), x_sorted, W1, W2)

    out = sc_gather(ffn_sorted, p_flat)
    return out.reshape((batch, n, dim))


# speedup vs baseline: 1.1207x; 1.0491x over previous
import functools

import jax
import jax.numpy as jnp
from jax import lax
from jax.experimental import pallas as pl
from jax.experimental.pallas import tpu as pltpu
from jax.experimental.pallas import tpu_sc as plsc

_DIM = 1024
_E = 8
_INNER = 4096
_T = 2048
_B = 128
_NBLK = _T // _B + _E
_P = _NBLK * _B
_EPS = 1e-06

_NC = 2
_NS = 16
_NW = _NC * _NS
_BPW = _T // _NW


def _gate_body(x_ref, wg_ref, bg_ref, xs_ref, p_ref, bmap_ref, act_ref):
    x = x_ref[...]
    logits = jnp.dot(x, wg_ref[...], preferred_element_type=jnp.float32)
    logits = logits + bg_ref[...]
    gs = jax.nn.softmax(logits, axis=-1)
    mx = jnp.max(gs, axis=1, keepdims=True)
    lane = lax.broadcasted_iota(jnp.int32, (_T, _E), 1)
    eid = jnp.min(jnp.where(gs == mx, lane, _E), axis=1, keepdims=True)
    onehot = (lane == eid).astype(jnp.float32)

    denom = jnp.sum(gs * onehot, axis=0, keepdims=True)
    counts_i = jnp.sum(onehot, axis=0, keepdims=True).astype(jnp.int32)
    pc = ((counts_i + (_B - 1)) // _B) * _B
    pc_f = pc.astype(jnp.float32)

    r8 = lax.broadcasted_iota(jnp.int32, (_E, _E), 0)
    c8 = lax.broadcasted_iota(jnp.int32, (_E, _E), 1)
    upper = (r8 < c8).astype(jnp.float32)
    off = jnp.dot(pc_f, upper, preferred_element_type=jnp.float32)

    ex = jnp.concatenate(
        [jnp.zeros((1, _E), jnp.float32), onehot[:-1]], axis=0)
    sh = 1
    while sh < _T:
        ex = ex + jnp.concatenate(
            [jnp.zeros((sh, _E), jnp.float32), ex[:-sh]], axis=0)
        sh *= 2
    rank = jnp.sum(ex * onehot, axis=1, keepdims=True)

    off_t = jnp.sum(off * onehot, axis=1, keepdims=True)
    denom_t = jnp.sum(denom * onehot, axis=1, keepdims=True)
    coeff = mx / (denom_t + _EPS) * float(_T)
    p_ref[...] = (off_t + rank).astype(jnp.int32)
    xs_ref[...] = x * coeff

    ends = off + pc_f
    rows_f = (lax.broadcasted_iota(jnp.int32, (_NBLK, _E), 0) * _B
              ).astype(jnp.float32)
    bmap_raw = jnp.sum((rows_f >= ends).astype(jnp.int32), axis=1,
                       keepdims=True)
    total = jnp.sum(pc)
    blk_lo = lax.broadcasted_iota(jnp.int32, (_NBLK, 1), 0) * _B
    act = (blk_lo < total).astype(jnp.int32)
    max_e = jnp.max(jnp.where(act == 1, bmap_raw, 0))
    bmap_ref[...] = jnp.minimum(bmap_raw, max_e)
    act_ref[...] = act


_KT = 2
_IK = _INNER // _KT
_S = _KT * _NBLK
_NSPLIT = 4


def _ffn_body(sched_ref, act_ref, x_ref, w1_hbm, w2_hbm, out_ref,
              acc_ref, xcache_ref, w1buf, w2buf, w1sem, w2sem):
    k = pl.program_id(0)
    i = pl.program_id(1)
    s = k * _NBLK + i
    chg = sched_ref[0, s]
    slot = sched_ref[1, s]
    cur_e = sched_ref[2, s]
    cur_k = sched_ref[3, s]
    nxt_e = sched_ref[4, s]
    nxt_k = sched_ref[5, s]
    has_n = sched_ref[6, s]

    def copies(e, kk, sl):
        cs = []
        r1 = _DIM // _NSPLIT
        r2 = _IK // _NSPLIT
        for j in range(_NSPLIT):
            cs.append(pltpu.make_async_copy(
                w1_hbm.at[e, pl.ds(j * r1, r1), pl.ds(kk * _IK, _IK)],
                w1buf.at[sl, pl.ds(j * r1, r1)], w1sem.at[sl, j]))
            cs.append(pltpu.make_async_copy(
                w2_hbm.at[e, pl.ds(kk * _IK + j * r2, r2), :],
                w2buf.at[sl, pl.ds(j * r2, r2)], w2sem.at[sl, j]))
        return cs

    @pl.when(s == 0)
    def _():
        for c in copies(cur_e, cur_k, slot):
            c.start()

    @pl.when(chg == 1)
    def _():
        for c in copies(cur_e, cur_k, slot):
            c.wait()

        @pl.when(has_n == 1)
        def _():
            for c in copies(nxt_e, nxt_k, 1 - slot):
                c.start()

    @pl.when(act_ref[i] == 1)
    def _():
        sl0 = pl.ds(i * _B, _B)

        @pl.when(k == 0)
        def _():
            xcache_ref[sl0, :] = x_ref[...]

        xv = jnp.where(k == 0, x_ref[...], xcache_ref[sl0, :])
        h = jnp.dot(xv, w1buf[slot],
                    preferred_element_type=jnp.float32)
        h = jnp.maximum(h, 0.0)
        part = jnp.dot(h, w2buf[slot], preferred_element_type=jnp.float32)
        sl = pl.ds(i * _B, _B)

        @pl.when(k == 0)
        def _():
            acc_ref[sl, :] = part

        @pl.when(k != 0)
        def _():
            acc_ref[sl, :] = acc_ref[sl, :] + part

        @pl.when(k == _KT - 1)
        def _():
            out_ref[...] = acc_ref[sl, :]


@functools.cache
def _make_sc_kernels():
    mesh = plsc.VectorSubcoreMesh(core_axis_name="c", subcore_axis_name="s")
    scratch = [
        pltpu.VMEM((_BPW,), jnp.int32),
        pltpu.VMEM((_BPW, _DIM), jnp.float32),
        pltpu.SemaphoreType.DMA,
    ]

    @functools.partial(
        pl.kernel,
        mesh=mesh,
        out_type=jax.ShapeDtypeStruct((_P, _DIM), jnp.float32),
        scratch_types=scratch,
    )
    def sc_scatter(x_hbm, p_hbm, out_hbm, idx_v, rows_v, sem):
        wid = lax.axis_index("s") * _NC + lax.axis_index("c")
        base = wid * _BPW
        pltpu.sync_copy(p_hbm.at[pl.ds(base, _BPW)], idx_v)
        pltpu.sync_copy(x_hbm.at[pl.ds(base, _BPW)], rows_v)
        pltpu.async_copy(rows_v, out_hbm.at[idx_v], sem).wait()

    @functools.partial(
        pl.kernel,
        mesh=mesh,
        out_type=jax.ShapeDtypeStruct((_T, _DIM), jnp.float32),
        scratch_types=scratch,
    )
    def sc_gather(src_hbm, p_hbm, out_hbm, idx_v, rows_v, sem):
        wid = lax.axis_index("s") * _NC + lax.axis_index("c")
        base = wid * _BPW
        pltpu.sync_copy(p_hbm.at[pl.ds(base, _BPW)], idx_v)
        pltpu.async_copy(src_hbm.at[idx_v], rows_v, sem).wait()
        pltpu.sync_copy(rows_v, out_hbm.at[pl.ds(base, _BPW)])

    return sc_scatter, sc_gather


def kernel(x, Wg, bg, W1, W2):
    batch, n, dim = x.shape
    x_flat = x.reshape((_T, _DIM))

    xs, p, bmap, act = pl.pallas_call(
        _gate_body,
        out_shape=(
            jax.ShapeDtypeStruct((_T, _DIM), jnp.float32),
            jax.ShapeDtypeStruct((_T, 1), jnp.int32),
            jax.ShapeDtypeStruct((_NBLK, 1), jnp.int32),
            jax.ShapeDtypeStruct((_NBLK, 1), jnp.int32),
        ),
    )(x_flat, Wg, bg.reshape((1, _E)))

    sc_scatter, sc_gather = _make_sc_kernels()
    p_flat = p.reshape((_T,))
    x_sorted = sc_scatter(xs, p_flat)

    bmap_flat = bmap.reshape((_NBLK,))
    e_step = jnp.tile(bmap_flat, _KT)
    k_step = jnp.repeat(jnp.arange(_KT, dtype=jnp.int32), _NBLK)
    idx = jnp.arange(_S, dtype=jnp.int32)
    prev_e = jnp.roll(e_step, 1)
    prev_k = jnp.roll(k_step, 1)
    chg = jnp.where(
        (idx == 0) | (e_step != prev_e) | (k_step != prev_k), 1, 0
    ).astype(jnp.int32)
    seg = jnp.cumsum(chg) - 1
    slot = (seg % 2).astype(jnp.int32)
    chg_idx = jnp.where(chg == 1, idx, _S)
    nc = jnp.min(
        jnp.where(idx[None, :] > idx[:, None], chg_idx[None, :], _S), axis=1)
    has_n = (nc < _S).astype(jnp.int32)
    nc_c = jnp.minimum(nc, _S - 1)
    sched = jnp.stack([
        chg, slot, e_step, k_step, e_step[nc_c], k_step[nc_c], has_n,
    ]).astype(jnp.int32)

    grid_spec = pltpu.PrefetchScalarGridSpec(
        num_scalar_prefetch=2,
        grid=(_KT, _NBLK),
        in_specs=[
            pl.BlockSpec(
                (_B, _DIM),
                lambda k, i, sc, ac: (
                    jnp.where((ac[i] == 1) & (k == 0), i, 0), 0)),
            pl.BlockSpec(memory_space=pltpu.MemorySpace.HBM),
            pl.BlockSpec(memory_space=pltpu.MemorySpace.HBM),
        ],
        out_specs=pl.BlockSpec(
            (_B, _DIM), lambda k, i, sc, ac: (jnp.where(k == _KT - 1, i, 0), 0)),
        scratch_shapes=[
            pltpu.VMEM((_P, _DIM), jnp.float32),
            pltpu.VMEM((_P, _DIM), jnp.float32),
            pltpu.VMEM((2, _DIM, _IK), jnp.float32),
            pltpu.VMEM((2, _IK, _DIM), jnp.float32),
            pltpu.SemaphoreType.DMA((2, _NSPLIT)),
            pltpu.SemaphoreType.DMA((2, _NSPLIT)),
        ],
    )
    ffn_sorted = pl.pallas_call(
        _ffn_body,
        grid_spec=grid_spec,
        out_shape=jax.ShapeDtypeStruct((_P, _DIM), jnp.float32),
        compiler_params=pltpu.CompilerParams(
            vmem_limit_bytes=63 * 1024 * 1024),
    )(sched, act.reshape((_NBLK,)), x_sorted, W1, W2)

    out = sc_gather(ffn_sorted, p_flat)
    return out.reshape((batch, n, dim))
